# Initial kernel scaffold; baseline (speedup 1.0000x reference)
import jax
import jax.numpy as jnp
from jax.experimental import pallas as pl


def kernel(token_ids, feature_table, special_embed, W, b):
    raise NotImplementedError("write your pallas kernel here")


# speedup vs baseline: 7.8678x; 7.8678x over previous
import functools
import math

import jax
import jax.numpy as jnp
from jax import lax
from jax.experimental import pallas as pl
from jax.experimental.pallas import tpu as pltpu
from jax.experimental.pallas import tpu_sc as plsc

D_MODEL = 64
FEAT_DIM = 37
VOCAB = 100001
SPECIAL_TOKEN_IDS = (0, 99998, 99999, 100000)

_GELU_C = math.sqrt(2.0 / math.pi)
_SCALE = math.sqrt(D_MODEL)


_BLK = 2048
_GRID = (VOCAB + _BLK - 1) // _BLK


def _table_body(ft_ref, wt_ref, b_ref, se_ref, out_ref):
    i = pl.program_id(0)
    feats = ft_ref[...]
    pe = jnp.dot(feats, wt_ref[...], preferred_element_type=jnp.float32)
    pe = pe + b_ref[...]
    pe = 0.5 * pe * (1.0 + jnp.tanh(_GELU_C * (pe + 0.044715 * pe * pe * pe)))
    pe = pe * _SCALE
    rows = i * _BLK + lax.broadcasted_iota(jnp.int32, (_BLK, 1), 0)
    for slot, tok in enumerate(SPECIAL_TOKEN_IDS):
        pe = jnp.where(rows == tok, se_ref[slot:slot + 1, :], pe)
    out_ref[...] = pe


def _build_table(feature_table, special_embed, W, b):
    wt = W.T
    b2 = b.reshape(1, D_MODEL)
    return pl.pallas_call(
        _table_body,
        grid=(_GRID,),
        in_specs=[
            pl.BlockSpec((_BLK, FEAT_DIM), lambda i: (i, 0)),
            pl.BlockSpec((FEAT_DIM, D_MODEL), lambda i: (0, 0)),
            pl.BlockSpec((1, D_MODEL), lambda i: (0, 0)),
            pl.BlockSpec((len(SPECIAL_TOKEN_IDS), D_MODEL), lambda i: (0, 0)),
        ],
        out_specs=pl.BlockSpec((_BLK, D_MODEL), lambda i: (i, 0)),
        out_shape=jax.ShapeDtypeStruct((VOCAB, D_MODEL), jnp.float32),
    )(feature_table, wt, b2, special_embed)


_L = 128
_IT = 16


def _make_gather(n_rows):
    info = plsc.get_sparse_core_info()
    nw = info.num_cores * info.num_subcores
    rows_per_w = n_rows // nw
    n_tiles = rows_per_w // _IT
    mesh = plsc.VectorSubcoreMesh(core_axis_name="c", subcore_axis_name="s")

    @functools.partial(
        pl.kernel,
        mesh=mesh,
        out_type=jax.ShapeDtypeStruct((n_rows * _L, D_MODEL), jnp.float32),
        scratch_types=[
            pltpu.VMEM((_IT, _L), jnp.int32),
            pltpu.VMEM((_L, D_MODEL), jnp.float32),
            pltpu.SemaphoreType.DMA,
        ],
        compiler_params=pltpu.CompilerParams(use_tc_tiling_on_sc=False),
    )
    def gather(tids_hbm, table_hbm, out_hbm, idx_v, row_v, sem):
        wid = lax.axis_index("s") * info.num_cores + lax.axis_index("c")
        wbase = wid * rows_per_w

        def tile_body(t, _):
            r0 = wbase + t * _IT
            pltpu.sync_copy(tids_hbm.at[pl.ds(r0, _IT)], idx_v)
            for j in range(_IT):
                pltpu.async_copy(table_hbm.at[idx_v.at[j]], row_v, sem).wait()
                pltpu.sync_copy(row_v, out_hbm.at[pl.ds((r0 + j) * _L, _L)])
            return 0

        lax.fori_loop(0, n_tiles, tile_body, 0)

    return gather


def kernel(token_ids, feature_table, special_embed, W, b):
    bsz, seq = token_ids.shape
    table = _build_table(feature_table, special_embed, W, b)
    n_rows = (bsz * seq) // _L
    tids2 = token_ids.reshape(n_rows, _L)
    flat = _make_gather(n_rows)(tids2, table)
    return flat.reshape(bsz, seq, D_MODEL)


# speedup vs baseline: 9.6770x; 1.2300x over previous
import functools
import math

import jax
import jax.numpy as jnp
from jax import lax
from jax.experimental import pallas as pl
from jax.experimental.pallas import tpu as pltpu
from jax.experimental.pallas import tpu_sc as plsc

D_MODEL = 64
FEAT_DIM = 37
VOCAB = 100001
SPECIAL_TOKEN_IDS = (0, 99998, 99999, 100000)

_GELU_C = math.sqrt(2.0 / math.pi)
_SCALE = math.sqrt(D_MODEL)


_BLK = 2048
_GRID = (VOCAB + _BLK - 1) // _BLK


def _table_body(ft_ref, wt_ref, b_ref, se_ref, out_ref):
    i = pl.program_id(0)
    feats = ft_ref[...]
    pe = jnp.dot(feats, wt_ref[...], preferred_element_type=jnp.float32)
    pe = pe + b_ref[...]
    pe = 0.5 * pe * (1.0 + jnp.tanh(_GELU_C * (pe + 0.044715 * pe * pe * pe)))
    pe = pe * _SCALE
    rows = i * _BLK + lax.broadcasted_iota(jnp.int32, (_BLK, 1), 0)
    for slot, tok in enumerate(SPECIAL_TOKEN_IDS):
        pe = jnp.where(rows == tok, se_ref[slot:slot + 1, :], pe)
    out_ref[...] = pe


def _build_table(feature_table, special_embed, W, b):
    wt = W.T
    b2 = b.reshape(1, D_MODEL)
    return pl.pallas_call(
        _table_body,
        grid=(_GRID,),
        in_specs=[
            pl.BlockSpec((_BLK, FEAT_DIM), lambda i: (i, 0)),
            pl.BlockSpec((FEAT_DIM, D_MODEL), lambda i: (0, 0)),
            pl.BlockSpec((1, D_MODEL), lambda i: (0, 0)),
            pl.BlockSpec((len(SPECIAL_TOKEN_IDS), D_MODEL), lambda i: (0, 0)),
        ],
        out_specs=pl.BlockSpec((_BLK, D_MODEL), lambda i: (i, 0)),
        out_shape=jax.ShapeDtypeStruct((VOCAB, D_MODEL), jnp.float32),
    )(feature_table, wt, b2, special_embed)


_L = 512


def _make_gather(n_rows):
    info = plsc.get_sparse_core_info()
    nw = info.num_cores * info.num_subcores
    n_chunks = n_rows // nw
    mesh = plsc.VectorSubcoreMesh(core_axis_name="c", subcore_axis_name="s")

    @functools.partial(
        pl.kernel,
        mesh=mesh,
        out_type=jax.ShapeDtypeStruct((n_rows * _L, D_MODEL), jnp.float32),
        scratch_types=[
            pltpu.VMEM((2, _L), jnp.int32),
            pltpu.VMEM((2, _L, D_MODEL), jnp.float32),
            pltpu.SemaphoreType.DMA((2,)),
            pltpu.SemaphoreType.DMA((2,)),
        ],
        compiler_params=pltpu.CompilerParams(use_tc_tiling_on_sc=False),
    )
    def gather(tids_hbm, table_hbm, out_hbm, idx_v, rows_v, gsem, osem):
        wid = lax.axis_index("s") * info.num_cores + lax.axis_index("c")
        wbase = wid * n_chunks

        def wait_gather(b):
            pltpu.make_async_copy(table_hbm.at[idx_v.at[b]], rows_v.at[b],
                                  gsem.at[b]).wait()

        def fire_scatter(k, b):
            pltpu.make_async_copy(
                rows_v.at[b],
                out_hbm.at[pl.ds((wbase + k) * _L, _L)],
                osem.at[b]).start()

        def wait_scatter(k, b):
            pltpu.make_async_copy(
                rows_v.at[b],
                out_hbm.at[pl.ds((wbase + k) * _L, _L)],
                osem.at[b]).wait()

        pltpu.sync_copy(tids_hbm.at[pl.ds(wbase * _L, _L)], idx_v.at[0])
        pltpu.make_async_copy(table_hbm.at[idx_v.at[0]], rows_v.at[0],
                              gsem.at[0]).start()

        def pair_body(t, _):
            for p in range(2):
                k = 2 * t + p
                nb = 1 - p
                @pl.when(k + 1 < n_chunks)
                def _():
                    pltpu.sync_copy(
                        tids_hbm.at[pl.ds((wbase + k + 1) * _L, _L)],
                        idx_v.at[nb])
                    @pl.when(k >= 1)
                    def _():
                        wait_scatter(k - 1, nb)
                    pltpu.make_async_copy(table_hbm.at[idx_v.at[nb]],
                                          rows_v.at[nb], gsem.at[nb]).start()
                wait_gather(p)
                fire_scatter(k, p)
            return 0

        lax.fori_loop(0, n_chunks // 2, pair_body, 0)
        wait_scatter(n_chunks - 2, 0)
        wait_scatter(n_chunks - 1, 1)

    return gather


def kernel(token_ids, feature_table, special_embed, W, b):
    bsz, seq = token_ids.shape
    table = _build_table(feature_table, special_embed, W, b)
    n_rows = (bsz * seq) // _L
    tids2 = token_ids.reshape(n_rows * _L)
    flat = _make_gather(n_rows)(tids2, table)
    return flat.reshape(bsz, seq, D_MODEL)
